# Initial kernel scaffold; baseline (speedup 1.0000x reference)
import functools

import jax
import jax.numpy as jnp
from jax import lax
from jax.experimental import pallas as pl
from jax.experimental.pallas import tpu as pltpu
from jax.experimental.pallas import tpu_sc as plsc

B = 4096
L = 200
D = 32
N = B * L
NC = 2
NS = 16
NW = NC * NS
PER_W = N // NW
R = 1600
NCH = PER_W // R
REPS = R // L
LANES = 16


def _emb_body(x_hbm, tab_hbm, pos_hbm, out_hbm, idx_v, rows_v, pos_v, sem):
    wid = lax.axis_index("s") * NC + lax.axis_index("c")
    base = wid * PER_W

    pltpu.sync_copy(pos_hbm, pos_v)

    def chunk_body(g, carry):
        off = base + g * R
        pltpu.sync_copy(x_hbm.at[pl.ds(off, R)], idx_v)
        pltpu.async_copy(tab_hbm.at[idx_v], rows_v, sem).wait()

        def rep_body(rep, c0):
            def row_body(j, c1):
                r = rep * L + j
                for h in range(D // LANES):
                    sl = pl.ds(h * LANES, LANES)
                    rows_v[r, sl] = rows_v[r, sl] + pos_v[j, sl]
                return c1
            return lax.fori_loop(0, L, row_body, c0)
        lax.fori_loop(0, REPS, rep_body, 0)

        pltpu.sync_copy(rows_v, out_hbm.at[pl.ds(off, R)])
        return carry

    lax.fori_loop(0, NCH, chunk_body, 0)


@jax.jit
def _emb(x_flat, table, pos):
    mesh = plsc.VectorSubcoreMesh(core_axis_name="c", subcore_axis_name="s")
    return pl.kernel(
        _emb_body,
        out_type=jax.ShapeDtypeStruct((N, D), jnp.float32),
        mesh=mesh,
        scratch_types=[
            pltpu.VMEM((R,), jnp.int32),
            pltpu.VMEM((R, D), jnp.float32),
            pltpu.VMEM((L, D), jnp.float32),
            pltpu.SemaphoreType.DMA,
        ],
    )(x_flat, table, pos)


def kernel(x, embedding_table, possitional_emb):
    out = _emb(x.reshape(-1).astype(jnp.int32), embedding_table,
               possitional_emb)
    return out.reshape(B, L, D)


# speedup vs baseline: 1.3203x; 1.3203x over previous
import functools

import jax
import jax.numpy as jnp
from jax import lax
from jax.experimental import pallas as pl
from jax.experimental.pallas import tpu as pltpu
from jax.experimental.pallas import tpu_sc as plsc

B = 4096
L = 200
D = 32
N = B * L
NC = 2
NS = 16
NW = NC * NS
PER_W = N // NW
R = 1600
NCH = PER_W // R
REPS = R // L
LANES = 16


def _emb_body(x_hbm, tab_hbm, pos_hbm, out_hbm, idx_v, rows_v, pos_v, sem):
    wid = lax.axis_index("s") * NC + lax.axis_index("c")
    base = wid * PER_W

    pltpu.sync_copy(pos_hbm, pos_v)

    def chunk_body(g, carry):
        off = base + g * R
        pltpu.sync_copy(x_hbm.at[pl.ds(off, R)], idx_v)
        pltpu.async_copy(tab_hbm.at[idx_v], rows_v, sem).wait()

        def rep_body(rep, c0):
            def row_body(j, c1):
                r = rep * L + j
                for h in range(D // LANES):
                    sl = pl.ds(h * LANES, LANES)
                    rows_v[r, sl] = rows_v[r, sl] + pos_v[j, sl]
                return c1
            return lax.fori_loop(0, L, row_body, c0)
        lax.fori_loop(0, REPS, rep_body, 0)

        pltpu.sync_copy(rows_v, out_hbm.at[pl.ds(off, R)])
        return carry

    lax.fori_loop(0, NCH, chunk_body, 0)


@jax.jit
def _emb(x_flat, table, pos):
    mesh = plsc.VectorSubcoreMesh(core_axis_name="c", subcore_axis_name="s")
    return pl.kernel(
        _emb_body,
        out_type=jax.ShapeDtypeStruct((N, D), jnp.float32),
        mesh=mesh,
        compiler_params=pltpu.CompilerParams(use_tc_tiling_on_sc=False),
        scratch_types=[
            pltpu.VMEM((R,), jnp.int32),
            pltpu.VMEM((R, D), jnp.float32),
            pltpu.VMEM((L, D), jnp.float32),
            pltpu.SemaphoreType.DMA,
        ],
    )(x_flat, table, pos)


def kernel(x, embedding_table, possitional_emb):
    out = _emb(x.reshape(-1).astype(jnp.int32), embedding_table,
               possitional_emb)
    return out.reshape(B, L, D)
